# baseline (speedup 1.0000x reference)
import functools

import jax
import jax.numpy as jnp
from jax.experimental import pallas as pl
from jax.experimental.pallas import tpu as pltpu

TOKENS = 32768
DIM = 768
N_EXPERTS = 8
CHUNK = 8192
NCHUNKS = TOKENS // CHUNK
NBUF = 2
SPLIT = 2
SUB = CHUNK // SPLIT


def _gate_kernel(x_hbm, w_ref, w_out_ref, idx_out_ref, xbuf, copy_sem):
    def copies(c):
        slot = c % NBUF
        return [
            pltpu.make_async_copy(
                x_hbm.at[pl.ds(c * CHUNK + j * SUB, SUB), :],
                xbuf.at[slot, pl.ds(j * SUB, SUB), :],
                copy_sem.at[slot, j],
            )
            for j in range(SPLIT)
        ]

    def start(c):
        for cp in copies(c):
            cp.start()

    for c in range(min(NBUF, NCHUNKS)):
        start(c)

    w = w_ref[...]
    for c in range(NCHUNKS):
        for cp in copies(c):
            cp.wait()
        s = jax.lax.dot_general(
            xbuf[c % NBUF], w,
            dimension_numbers=(((1,), (1,)), ((), ())),
            preferred_element_type=jnp.float32)
        if c + NBUF < NCHUNKS:
            start(c + NBUF)
        st = s.T
        m = jnp.max(st, axis=0, keepdims=True)
        denom = jnp.sum(jnp.exp(st - m), axis=0, keepdims=True)
        w_out_ref[pl.ds(c * CHUNK, CHUNK)] = (1.0 / denom).reshape(CHUNK)
        idx_out_ref[pl.ds(c * CHUNK, CHUNK)] = (
            jnp.argmax(st, axis=0).reshape(CHUNK).astype(jnp.int32))


@jax.jit
def kernel(x, weight):
    weights, indices = pl.pallas_call(
        _gate_kernel,
        in_specs=[
            pl.BlockSpec(memory_space=pltpu.HBM),
            pl.BlockSpec(memory_space=pltpu.VMEM),
        ],
        out_specs=[
            pl.BlockSpec(memory_space=pltpu.VMEM),
            pl.BlockSpec(memory_space=pltpu.VMEM),
        ],
        out_shape=[
            jax.ShapeDtypeStruct((TOKENS,), jnp.float32),
            jax.ShapeDtypeStruct((TOKENS,), jnp.int32),
        ],
        scratch_shapes=[
            pltpu.VMEM((NBUF, CHUNK, DIM), jnp.float32),
            pltpu.SemaphoreType.DMA((NBUF, SPLIT)),
        ],
    )(x, weight)
    return weights.reshape(TOKENS, 1), indices.reshape(TOKENS, 1)


# speedup vs baseline: 1.0808x; 1.0808x over previous
import functools

import jax
import jax.numpy as jnp
from jax.experimental import pallas as pl

TOKENS = 32768
DIM = 768
N_EXPERTS = 8
BLOCK = 8192


def _gate_kernel(x_ref, w_ref, w_out_ref, idx_out_ref):
    s = jax.lax.dot_general(
        x_ref[...], w_ref[...],
        dimension_numbers=(((1,), (1,)), ((), ())),
        preferred_element_type=jnp.float32)
    st = s.T
    m = jnp.max(st, axis=0, keepdims=True)
    denom = jnp.sum(jnp.exp(st - m), axis=0, keepdims=True)
    w_out_ref[...] = (1.0 / denom).reshape(BLOCK)
    idx_out_ref[...] = jnp.argmax(st, axis=0).reshape(BLOCK).astype(jnp.int32)


@jax.jit
def kernel(x, weight):
    grid = (TOKENS // BLOCK,)
    weights, indices = pl.pallas_call(
        _gate_kernel,
        grid=grid,
        in_specs=[
            pl.BlockSpec((BLOCK, DIM), lambda i: (i, 0)),
            pl.BlockSpec((N_EXPERTS, DIM), lambda i: (0, 0)),
        ],
        out_specs=[
            pl.BlockSpec((BLOCK,), lambda i: (i,)),
            pl.BlockSpec((BLOCK,), lambda i: (i,)),
        ],
        out_shape=[
            jax.ShapeDtypeStruct((TOKENS,), jnp.float32),
            jax.ShapeDtypeStruct((TOKENS,), jnp.int32),
        ],
    )(x, weight)
    return weights.reshape(TOKENS, 1), indices.reshape(TOKENS, 1)


# speedup vs baseline: 1.1575x; 1.0710x over previous
import functools

import jax
import jax.numpy as jnp
from jax.experimental import pallas as pl

TOKENS = 32768
DIM = 768
N_EXPERTS = 8
BLOCK = 4096


def _gate_kernel(x_ref, w_ref, w_out_ref, idx_out_ref):
    s = jax.lax.dot_general(
        x_ref[...], w_ref[...],
        dimension_numbers=(((1,), (1,)), ((), ())),
        preferred_element_type=jnp.float32)
    st = s.T
    m = jnp.max(st, axis=0, keepdims=True)
    denom = jnp.sum(jnp.exp(st - m), axis=0, keepdims=True)
    w_out_ref[...] = (1.0 / denom).reshape(BLOCK)
    idx_out_ref[...] = jnp.argmax(st, axis=0).reshape(BLOCK).astype(jnp.int32)


@jax.jit
def kernel(x, weight):
    grid = (TOKENS // BLOCK,)
    weights, indices = pl.pallas_call(
        _gate_kernel,
        grid=grid,
        in_specs=[
            pl.BlockSpec((BLOCK, DIM), lambda i: (i, 0)),
            pl.BlockSpec((N_EXPERTS, DIM), lambda i: (0, 0)),
        ],
        out_specs=[
            pl.BlockSpec((BLOCK,), lambda i: (i,)),
            pl.BlockSpec((BLOCK,), lambda i: (i,)),
        ],
        out_shape=[
            jax.ShapeDtypeStruct((TOKENS,), jnp.float32),
            jax.ShapeDtypeStruct((TOKENS,), jnp.int32),
        ],
    )(x, weight)
    return weights.reshape(TOKENS, 1), indices.reshape(TOKENS, 1)
